# Initial kernel scaffold; baseline (speedup 1.0000x reference)
import jax
import jax.numpy as jnp
from jax.experimental import pallas as pl


def kernel(scaled_atomic_energy, atom_type, scale, shift):
    raise NotImplementedError("write your pallas kernel here")


# speedup vs baseline: 1.0196x; 1.0196x over previous
import functools

import jax
import jax.numpy as jnp
from jax import lax
from jax.experimental import pallas as pl
from jax.experimental.pallas import tpu as pltpu
from jax.experimental.pallas import tpu_sc as plsc

_NC = 2
_NS = 16
_NW = _NC * _NS
_L = 16


def _make_kernel(n):
    chunk = (n // (_NW * _L)) * _L
    tail_vecs = (n - _NW * chunk) // _L
    assert chunk > 0 and _NW * chunk + tail_vecs * _L == n
    tail_base = _NW * chunk

    mesh = plsc.VectorSubcoreMesh(core_axis_name="c", subcore_axis_name="s")

    @functools.partial(
        pl.kernel,
        out_type=jax.ShapeDtypeStruct((n,), jnp.float32),
        mesh=mesh,
        compiler_params=pltpu.CompilerParams(needs_layout_passes=False),
        scratch_types=[
            pltpu.VMEM((chunk,), jnp.float32),
            pltpu.VMEM((chunk,), jnp.int32),
            pltpu.VMEM((chunk,), jnp.float32),
            pltpu.VMEM((_L,), jnp.float32),
            pltpu.VMEM((_L,), jnp.float32),
            pltpu.VMEM((_L,), jnp.float32),
            pltpu.VMEM((_L,), jnp.int32),
            pltpu.VMEM((_L,), jnp.float32),
        ],
    )
    def rescale(x_hbm, t_hbm, scale_hbm, shift_hbm, out_hbm,
                x_v, t_v, o_v, scale_v, shift_v, xt_v, tt_v, ot_v):
        wid = lax.axis_index("s") * _NC + lax.axis_index("c")
        base = wid * chunk

        pltpu.sync_copy(scale_hbm, scale_v)
        pltpu.sync_copy(shift_hbm, shift_v)
        pltpu.sync_copy(x_hbm.at[pl.ds(base, chunk)], x_v)
        pltpu.sync_copy(t_hbm.at[pl.ds(base, chunk)], t_v)

        def body(i, carry):
            off = i * _L
            idx = t_v[pl.ds(off, _L)]
            xv = x_v[pl.ds(off, _L)]
            s = plsc.load_gather(scale_v, [idx])
            b = plsc.load_gather(shift_v, [idx])
            o_v[pl.ds(off, _L)] = xv * s + b
            return carry

        lax.fori_loop(0, chunk // _L, body, 0)
        pltpu.sync_copy(o_v, out_hbm.at[pl.ds(base, chunk)])

        if tail_vecs:
            @pl.when(wid < tail_vecs)
            def _():
                tb = tail_base + wid * _L
                pltpu.sync_copy(x_hbm.at[pl.ds(tb, _L)], xt_v)
                pltpu.sync_copy(t_hbm.at[pl.ds(tb, _L)], tt_v)
                idx = tt_v[...]
                s = plsc.load_gather(scale_v, [idx])
                b = plsc.load_gather(shift_v, [idx])
                ot_v[...] = xt_v[...] * s + b
                pltpu.sync_copy(ot_v, out_hbm.at[pl.ds(tb, _L)])

    return rescale


def kernel(scaled_atomic_energy, atom_type, scale, shift):
    n = scaled_atomic_energy.shape[0]
    x = scaled_atomic_energy.reshape(n)
    t = atom_type.astype(jnp.int32)
    out = _make_kernel(n)(x, t, scale, shift)
    return out.reshape(n, 1)


# speedup vs baseline: 1.1394x; 1.1174x over previous
import functools

import jax
import jax.numpy as jnp
from jax import lax
from jax.experimental import pallas as pl
from jax.experimental.pallas import tpu as pltpu
from jax.experimental.pallas import tpu_sc as plsc

_NC = 2
_NS = 16
_NW = _NC * _NS
_L = 16


def _make_kernel(n):
    chunk = (n // (_NW * _L)) * _L
    tail_vecs = (n - _NW * chunk) // _L
    assert chunk > 0 and _NW * chunk + tail_vecs * _L == n
    tail_base = _NW * chunk

    mesh = plsc.VectorSubcoreMesh(core_axis_name="c", subcore_axis_name="s")

    @functools.partial(
        pl.kernel,
        out_type=jax.ShapeDtypeStruct((n,), jnp.float32),
        mesh=mesh,
        compiler_params=pltpu.CompilerParams(needs_layout_passes=False),
        scratch_types=[
            pltpu.VMEM((chunk,), jnp.float32),
            pltpu.VMEM((chunk,), jnp.int32),
            pltpu.VMEM((chunk,), jnp.float32),
            pltpu.VMEM((_L,), jnp.float32),
            pltpu.VMEM((_L,), jnp.float32),
            pltpu.VMEM((_L,), jnp.float32),
            pltpu.VMEM((_L,), jnp.int32),
            pltpu.VMEM((_L,), jnp.float32),
            pltpu.SemaphoreType.DMA,
        ],
    )
    def rescale(x_hbm, t_hbm, scale_hbm, shift_hbm, out_hbm,
                x_v, t_v, o_v, scale_v, shift_v, xt_v, tt_v, ot_v, sem):
        wid = lax.axis_index("s") * _NC + lax.axis_index("c")
        base = wid * chunk

        copies = [
            pltpu.async_copy(scale_hbm, scale_v, sem),
            pltpu.async_copy(shift_hbm, shift_v, sem),
            pltpu.async_copy(x_hbm.at[pl.ds(base, chunk)], x_v, sem),
            pltpu.async_copy(t_hbm.at[pl.ds(base, chunk)], t_v, sem),
        ]
        if tail_vecs:
            @pl.when(wid < tail_vecs)
            def _():
                tb = tail_base + wid * _L
                cx = pltpu.async_copy(x_hbm.at[pl.ds(tb, _L)], xt_v, sem)
                ct = pltpu.async_copy(t_hbm.at[pl.ds(tb, _L)], tt_v, sem)
                cx.wait()
                ct.wait()
        for c in copies:
            c.wait()

        @plsc.parallel_loop(0, chunk, step=_L, unroll=8)
        def _(off):
            idx = t_v[pl.ds(off, _L)]
            xv = x_v[pl.ds(off, _L)]
            s = plsc.load_gather(scale_v, [idx])
            b = plsc.load_gather(shift_v, [idx])
            o_v[pl.ds(off, _L)] = xv * s + b

        if tail_vecs:
            @pl.when(wid < tail_vecs)
            def _():
                tb = tail_base + wid * _L
                idx = tt_v[...]
                s = plsc.load_gather(scale_v, [idx])
                b = plsc.load_gather(shift_v, [idx])
                ot_v[...] = xt_v[...] * s + b
                pltpu.sync_copy(ot_v, out_hbm.at[pl.ds(tb, _L)])

        pltpu.sync_copy(o_v, out_hbm.at[pl.ds(base, chunk)])

    return rescale


def kernel(scaled_atomic_energy, atom_type, scale, shift):
    n = scaled_atomic_energy.shape[0]
    x = scaled_atomic_energy.reshape(n)
    t = atom_type.astype(jnp.int32)
    out = _make_kernel(n)(x, t, scale, shift)
    return out.reshape(n, 1)
